# Initial kernel scaffold; baseline (speedup 1.0000x reference)
import jax
import jax.numpy as jnp
from jax.experimental import pallas as pl


def kernel(x, edge_index, W1, b1, W2, b2):
    raise NotImplementedError("write your pallas kernel here")


# speedup vs baseline: 30.3795x; 30.3795x over previous
import functools

import jax
import jax.numpy as jnp
from jax import lax
from jax.experimental import pallas as pl
from jax.experimental.pallas import tpu as pltpu
from jax.experimental.pallas import tpu_sc as plsc

N = 10000
E = 320000
D = 128
H = 16
C = 2

NC = 2
NS = 16
NW = NC * NS
EPW = E // NW
B = 80
NBLK = EPW // B
NPAD = 10240
RPT = NPAD // NS

_MESH = plsc.VectorSubcoreMesh(core_axis_name="c", subcore_axis_name="s")
_LANES = 16


@functools.partial(
    pl.kernel,
    out_type=jax.ShapeDtypeStruct((NC, NPAD), jnp.float32),
    mesh=_MESH,
    scratch_types=[
        pltpu.VMEM((NBLK, B), jnp.int32),
        pltpu.VMEM((B,), jnp.float32),
        pltpu.VMEM((RPT,), jnp.float32),
        pltpu.VMEM_SHARED((NPAD,), jnp.float32),
    ],
    compiler_params=pltpu.CompilerParams(use_tc_tiling_on_sc=False),
)
def _sc_degree(dst_hbm, out_hbm, dst_v, ones_v, zv, acc_sh):
    c = lax.axis_index("c")
    s = lax.axis_index("s")
    wid = s * NC + c

    for i in range(B // _LANES):
        ones_v[pl.ds(i * _LANES, _LANES)] = jnp.ones((_LANES,), jnp.float32)

    def zbody(i, _):
        zv[pl.ds(i * _LANES, _LANES)] = jnp.zeros((_LANES,), jnp.float32)
        return 0
    lax.fori_loop(0, RPT // _LANES, zbody, 0)

    pltpu.sync_copy(zv, acc_sh.at[pl.ds(s * RPT, RPT)])
    pltpu.sync_copy(dst_hbm.at[wid], dst_v)
    plsc.subcore_barrier()

    def body(j, _):
        pltpu.sync_copy(ones_v, acc_sh.at[dst_v.at[j]], add=True)
        return 0
    lax.fori_loop(0, NBLK, body, 0)

    plsc.subcore_barrier()
    pltpu.sync_copy(acc_sh.at[pl.ds(s * RPT, RPT)],
                    out_hbm.at[c, pl.ds(s * RPT, RPT)])


@functools.partial(
    pl.kernel,
    out_type=jax.ShapeDtypeStruct((NC, NPAD, H), jnp.float32),
    mesh=_MESH,
    scratch_types=[
        pltpu.VMEM((NBLK, B), jnp.int32),
        pltpu.VMEM((NBLK, B), jnp.int32),
        pltpu.VMEM((B, H), jnp.float32),
        pltpu.VMEM((RPT, H), jnp.float32),
        pltpu.VMEM_SHARED((NPAD, H), jnp.float32),
        pltpu.SemaphoreType.DMA,
    ],
    compiler_params=pltpu.CompilerParams(use_tc_tiling_on_sc=False),
)
def _sc_propagate(g_hbm, src_hbm, dst_hbm, out_hbm,
                  src_v, dst_v, rows_v, zv, acc_sh, sem):
    c = lax.axis_index("c")
    s = lax.axis_index("s")
    wid = s * NC + c

    def zbody(i, _):
        zv[i] = jnp.zeros((_LANES,), jnp.float32)
        return 0
    lax.fori_loop(0, RPT, zbody, 0)

    pltpu.sync_copy(zv, acc_sh.at[pl.ds(s * RPT, RPT)])
    pltpu.sync_copy(src_hbm.at[wid], src_v)
    pltpu.sync_copy(dst_hbm.at[wid], dst_v)
    plsc.subcore_barrier()

    def body(j, _):
        pltpu.async_copy(g_hbm.at[src_v.at[j]], rows_v, sem).wait()
        pltpu.sync_copy(rows_v, acc_sh.at[dst_v.at[j]], add=True)
        return 0
    lax.fori_loop(0, NBLK, body, 0)

    plsc.subcore_barrier()
    pltpu.sync_copy(acc_sh.at[pl.ds(s * RPT, RPT)],
                    out_hbm.at[c, pl.ds(s * RPT, RPT)])


def _tc_scale_in(xp_ref, w1_ref, degp_ref, g1_ref, dis_ref):
    deg = 1.0 + degp_ref[0] + degp_ref[1]
    dis = lax.rsqrt(deg)
    h = jnp.dot(xp_ref[...], w1_ref[...], preferred_element_type=jnp.float32)
    dis_ref[...] = dis
    g1_ref[...] = h * dis[:, None]


def _tc_mid(s1_ref, g1_ref, dis_ref, b1_ref, g2_ref):
    dis = dis_ref[...]
    tot = s1_ref[0] + s1_ref[1] + g1_ref[...]
    r = jnp.maximum(tot * dis[:, None] + b1_ref[...][None, :], 0.0)
    g2_ref[...] = r * dis[:, None]


def _tc_out(s2_ref, g2_ref, dis_ref, w2_ref, b2_ref, o_ref):
    dis = dis_ref[...]
    agg = (s2_ref[0] + s2_ref[1] + g2_ref[...]) * dis[:, None]
    o = jnp.dot(agg, w2_ref[...], preferred_element_type=jnp.float32)
    o = o + b2_ref[...][None, :]
    m = jnp.max(o, axis=1, keepdims=True)
    lse = m + jnp.log(jnp.sum(jnp.exp(o - m), axis=1, keepdims=True))
    o_ref[...] = (o - lse)[:N]


def kernel(x, edge_index, W1, b1, W2, b2):
    src3 = edge_index[0].reshape(NW, NBLK, B)
    dst3 = edge_index[1].reshape(NW, NBLK, B)
    xp = jnp.pad(x, ((0, NPAD - N), (0, 0)))

    degp = _sc_degree(dst3)

    g1, dis = pl.pallas_call(
        _tc_scale_in,
        out_shape=(
            jax.ShapeDtypeStruct((NPAD, H), jnp.float32),
            jax.ShapeDtypeStruct((NPAD,), jnp.float32),
        ),
    )(xp, W1, degp)

    s1 = _sc_propagate(g1, src3, dst3)

    g2 = pl.pallas_call(
        _tc_mid,
        out_shape=jax.ShapeDtypeStruct((NPAD, H), jnp.float32),
    )(s1, g1, dis, b1)

    s2 = _sc_propagate(g2, src3, dst3)

    out = pl.pallas_call(
        _tc_out,
        out_shape=jax.ShapeDtypeStruct((N, C), jnp.float32),
    )(s2, g2, dis, W2, b2)

    return out


# speedup vs baseline: 40.5437x; 1.3346x over previous
import functools

import jax
import jax.numpy as jnp
from jax import lax
from jax.experimental import pallas as pl
from jax.experimental.pallas import tpu as pltpu
from jax.experimental.pallas import tpu_sc as plsc

N = 10000
E = 320000
D = 128
H = 16
C = 2

NC = 2
NS = 16
NW = NC * NS
B = 128
NBLK = 80
EPW = B * NBLK
EP = NW * EPW
NPAD = 10240
DUMMY = NPAD - 1
RPT = NPAD // NS
RING = 5

_MESH = plsc.VectorSubcoreMesh(core_axis_name="c", subcore_axis_name="s")
_LANES = 16


@functools.partial(
    pl.kernel,
    out_type=jax.ShapeDtypeStruct((NC, NPAD), jnp.float32),
    mesh=_MESH,
    scratch_types=[
        pltpu.VMEM((NBLK, B), jnp.int32),
        pltpu.VMEM((B,), jnp.float32),
        pltpu.VMEM((RPT,), jnp.float32),
        pltpu.VMEM_SHARED((NPAD,), jnp.float32),
        pltpu.SemaphoreType.DMA,
    ],
    compiler_params=pltpu.CompilerParams(use_tc_tiling_on_sc=False),
)
def _sc_degree(dst_hbm, out_hbm, dst_v, ones_v, zv, acc_sh, sem):
    c = lax.axis_index("c")
    s = lax.axis_index("s")
    wid = s * NC + c

    for i in range(B // _LANES):
        ones_v[pl.ds(i * _LANES, _LANES)] = jnp.ones((_LANES,), jnp.float32)

    def zbody(i, _):
        zv[pl.ds(i * _LANES, _LANES)] = jnp.zeros((_LANES,), jnp.float32)
        return 0
    lax.fori_loop(0, RPT // _LANES, zbody, 0)

    pltpu.sync_copy(zv, acc_sh.at[pl.ds(s * RPT, RPT)])
    pltpu.sync_copy(dst_hbm.at[wid], dst_v)
    plsc.subcore_barrier()

    def body(j, _):
        pltpu.async_copy(ones_v, acc_sh.at[dst_v.at[j]], sem, add=True)

        @pl.when(j >= 2)
        def _wait():
            pltpu.make_async_copy(ones_v, acc_sh.at[dst_v.at[0]], sem).wait()
        return 0
    lax.fori_loop(0, NBLK, body, 0)
    for _ in range(2):
        pltpu.make_async_copy(ones_v, acc_sh.at[dst_v.at[0]], sem).wait()

    plsc.subcore_barrier()
    pltpu.sync_copy(acc_sh.at[pl.ds(s * RPT, RPT)],
                    out_hbm.at[c, pl.ds(s * RPT, RPT)])


@functools.partial(
    pl.kernel,
    out_type=jax.ShapeDtypeStruct((NC, NPAD, H), jnp.float32),
    mesh=_MESH,
    scratch_types=[
        pltpu.VMEM((NBLK, B), jnp.int32),
        pltpu.VMEM((NBLK, B), jnp.int32),
        pltpu.VMEM((RING, B, H), jnp.float32),
        pltpu.VMEM((RPT, H), jnp.float32),
        pltpu.VMEM_SHARED((NPAD, H), jnp.float32),
        [pltpu.SemaphoreType.DMA] * RING,
        [pltpu.SemaphoreType.DMA] * RING,
    ],
    compiler_params=pltpu.CompilerParams(use_tc_tiling_on_sc=False),
)
def _sc_propagate(g_hbm, src_hbm, dst_hbm, out_hbm,
                  src_v, dst_v, rows_v, zv, acc_sh, sg, ss):
    c = lax.axis_index("c")
    s = lax.axis_index("s")
    wid = s * NC + c

    def zbody(i, _):
        zv[i] = jnp.zeros((_LANES,), jnp.float32)
        return 0
    lax.fori_loop(0, RPT, zbody, 0)

    pltpu.sync_copy(zv, acc_sh.at[pl.ds(s * RPT, RPT)])
    pltpu.sync_copy(src_hbm.at[wid], src_v)
    pltpu.sync_copy(dst_hbm.at[wid], dst_v)
    plsc.subcore_barrier()

    def gather_wait(b):
        pltpu.make_async_copy(
            g_hbm.at[src_v.at[0]], rows_v.at[b], sg[b]).wait()

    def scatter_wait(b):
        pltpu.make_async_copy(
            rows_v.at[b], acc_sh.at[dst_v.at[0]], ss[b]).wait()

    for b in range(RING - 1):
        pltpu.async_copy(g_hbm.at[src_v.at[b]], rows_v.at[b], sg[b])

    def body(i, _):
        for t in range(RING):
            j = i * RING + t
            nb = (t + RING - 1) % RING
            nj = j + RING - 1

            @pl.when(j > 0)
            def _ws():
                scatter_wait(nb)

            @pl.when(nj < NBLK)
            def _sg():
                pltpu.async_copy(
                    g_hbm.at[src_v.at[nj]], rows_v.at[nb], sg[nb])

            gather_wait(t)
            pltpu.async_copy(
                rows_v.at[t], acc_sh.at[dst_v.at[j]], ss[t], add=True)
        return 0
    lax.fori_loop(0, NBLK // RING, body, 0)
    scatter_wait((NBLK - 1) % RING)

    plsc.subcore_barrier()
    pltpu.sync_copy(acc_sh.at[pl.ds(s * RPT, RPT)],
                    out_hbm.at[c, pl.ds(s * RPT, RPT)])


def _tc_scale_in(xp_ref, w1_ref, degp_ref, g1_ref, dis_ref):
    deg = 1.0 + degp_ref[0] + degp_ref[1]
    dis = lax.rsqrt(deg)
    h = jnp.dot(xp_ref[...], w1_ref[...], preferred_element_type=jnp.float32)
    dis_ref[...] = dis
    g1_ref[...] = h * dis[:, None]


def _tc_mid(s1_ref, g1_ref, dis_ref, b1_ref, g2_ref):
    dis = dis_ref[...]
    tot = s1_ref[0] + s1_ref[1] + g1_ref[...]
    r = jnp.maximum(tot * dis[:, None] + b1_ref[...][None, :], 0.0)
    g2_ref[...] = r * dis[:, None]


def _tc_out(s2_ref, g2_ref, dis_ref, w2_ref, b2_ref, o_ref):
    dis = dis_ref[...]
    agg = (s2_ref[0] + s2_ref[1] + g2_ref[...]) * dis[:, None]
    o = jnp.dot(agg, w2_ref[...], preferred_element_type=jnp.float32)
    o = o + b2_ref[...][None, :]
    m = jnp.max(o, axis=1, keepdims=True)
    lse = m + jnp.log(jnp.sum(jnp.exp(o - m), axis=1, keepdims=True))
    o_ref[...] = (o - lse)[:N]


def kernel(x, edge_index, W1, b1, W2, b2):
    pad = jnp.full((EP - E,), DUMMY, dtype=edge_index.dtype)
    src3 = jnp.concatenate([edge_index[0], pad]).reshape(NW, NBLK, B)
    dst3 = jnp.concatenate([edge_index[1], pad]).reshape(NW, NBLK, B)
    xp = jnp.pad(x, ((0, NPAD - N), (0, 0)))

    degp = _sc_degree(dst3)

    g1, dis = pl.pallas_call(
        _tc_scale_in,
        out_shape=(
            jax.ShapeDtypeStruct((NPAD, H), jnp.float32),
            jax.ShapeDtypeStruct((NPAD,), jnp.float32),
        ),
    )(xp, W1, degp)

    s1 = _sc_propagate(g1, src3, dst3)

    g2 = pl.pallas_call(
        _tc_mid,
        out_shape=jax.ShapeDtypeStruct((NPAD, H), jnp.float32),
    )(s1, g1, dis, b1)

    s2 = _sc_propagate(g2, src3, dst3)

    out = pl.pallas_call(
        _tc_out,
        out_shape=jax.ShapeDtypeStruct((N, C), jnp.float32),
    )(s2, g2, dis, W2, b2)

    return out


# speedup vs baseline: 71.7294x; 1.7692x over previous
import functools

import jax
import jax.numpy as jnp
from jax import lax
from jax.experimental import pallas as pl
from jax.experimental.pallas import tpu as pltpu
from jax.experimental.pallas import tpu_sc as plsc

N = 10000
E = 320000
D = 128
H = 16
C = 2

NC = 2
NS = 16
NW = NC * NS
EPT = E // NW
B = 128
NBLK = EPT // B
TAIL = EPT - NBLK * B
RING = 6
NPAD = 10240
RPT = NPAD // NS

_MESH = plsc.VectorSubcoreMesh(core_axis_name="c", subcore_axis_name="s")
_LANES = 16


@functools.partial(
    pl.kernel,
    out_type=jax.ShapeDtypeStruct((NC, NPAD), jnp.float32),
    mesh=_MESH,
    scratch_types=[
        pltpu.VMEM((EPT,), jnp.int32),
        pltpu.VMEM((B,), jnp.float32),
        pltpu.VMEM((RPT,), jnp.float32),
        pltpu.VMEM_SHARED((NPAD,), jnp.float32),
        pltpu.SemaphoreType.DMA,
    ],
    compiler_params=pltpu.CompilerParams(use_tc_tiling_on_sc=False),
)
def _sc_degree(edge_hbm, out_hbm, dst_v, ones_v, zv, acc_sh, sem):
    c = lax.axis_index("c")
    s = lax.axis_index("s")
    wid = s * NC + c

    for i in range(B // _LANES):
        ones_v[pl.ds(i * _LANES, _LANES)] = jnp.ones((_LANES,), jnp.float32)

    def zbody(i, _):
        zv[pl.ds(i * _LANES, _LANES)] = jnp.zeros((_LANES,), jnp.float32)
        return 0
    lax.fori_loop(0, RPT // _LANES, zbody, 0)

    pltpu.sync_copy(zv, acc_sh.at[pl.ds(s * RPT, RPT)])
    pltpu.sync_copy(edge_hbm.at[1, pl.ds(wid * EPT, EPT)], dst_v)
    plsc.subcore_barrier()

    def body(j, _):
        pltpu.async_copy(
            ones_v, acc_sh.at[dst_v.at[pl.ds(j * B, B)]], sem, add=True)

        @pl.when(j >= 2)
        def _wait():
            pltpu.make_async_copy(
                ones_v, acc_sh.at[dst_v.at[pl.ds(0, B)]], sem).wait()
        return 0
    lax.fori_loop(0, NBLK, body, 0)
    for _ in range(2):
        pltpu.make_async_copy(
            ones_v, acc_sh.at[dst_v.at[pl.ds(0, B)]], sem).wait()
    pltpu.sync_copy(ones_v.at[pl.ds(0, TAIL)],
                    acc_sh.at[dst_v.at[pl.ds(NBLK * B, TAIL)]], add=True)

    plsc.subcore_barrier()
    pltpu.sync_copy(acc_sh.at[pl.ds(s * RPT, RPT)],
                    out_hbm.at[c, pl.ds(s * RPT, RPT)])


@functools.partial(
    pl.kernel,
    out_type=jax.ShapeDtypeStruct((NC, NPAD, H), jnp.float32),
    mesh=_MESH,
    scratch_types=[
        pltpu.VMEM((EPT,), jnp.int32),
        pltpu.VMEM((EPT,), jnp.int32),
        pltpu.VMEM((RING, B, H), jnp.float32),
        pltpu.VMEM((TAIL, H), jnp.float32),
        pltpu.VMEM((RPT, H), jnp.float32),
        pltpu.VMEM_SHARED((NPAD, H), jnp.float32),
        [pltpu.SemaphoreType.DMA] * RING,
        [pltpu.SemaphoreType.DMA] * RING,
        pltpu.SemaphoreType.DMA,
    ],
    compiler_params=pltpu.CompilerParams(use_tc_tiling_on_sc=False),
)
def _sc_propagate(g_hbm, edge_hbm, out_hbm,
                  src_v, dst_v, rows_v, trow_v, zv, acc_sh, sg, ss, st):
    c = lax.axis_index("c")
    s = lax.axis_index("s")
    wid = s * NC + c

    def zbody(i, _):
        zv[i] = jnp.zeros((_LANES,), jnp.float32)
        return 0
    lax.fori_loop(0, RPT, zbody, 0)

    pltpu.sync_copy(zv, acc_sh.at[pl.ds(s * RPT, RPT)])
    pltpu.sync_copy(edge_hbm.at[0, pl.ds(wid * EPT, EPT)], src_v)
    pltpu.sync_copy(edge_hbm.at[1, pl.ds(wid * EPT, EPT)], dst_v)
    plsc.subcore_barrier()

    def gather_start(j, b):
        pltpu.async_copy(
            g_hbm.at[src_v.at[pl.ds(j * B, B)]], rows_v.at[b], sg[b])

    def gather_wait(b):
        pltpu.make_async_copy(
            g_hbm.at[src_v.at[pl.ds(0, B)]], rows_v.at[b], sg[b]).wait()

    def scatter_wait(b):
        pltpu.make_async_copy(
            rows_v.at[b], acc_sh.at[dst_v.at[pl.ds(0, B)]], ss[b]).wait()

    for b in range(RING - 1):
        gather_start(b, b)

    def body(i, _):
        for t in range(RING):
            j = i * RING + t
            nb = (t + RING - 1) % RING
            nj = j + RING - 1

            @pl.when(j > 0)
            def _ws():
                scatter_wait(nb)

            @pl.when(nj < NBLK)
            def _sg():
                gather_start(nj, nb)

            gather_wait(t)
            pltpu.async_copy(
                rows_v.at[t], acc_sh.at[dst_v.at[pl.ds(j * B, B)]],
                ss[t], add=True)
        return 0
    lax.fori_loop(0, NBLK // RING, body, 0)
    scatter_wait((NBLK - 1) % RING)

    pltpu.async_copy(
        g_hbm.at[src_v.at[pl.ds(NBLK * B, TAIL)]], trow_v, st).wait()
    pltpu.sync_copy(
        trow_v, acc_sh.at[dst_v.at[pl.ds(NBLK * B, TAIL)]], add=True)

    plsc.subcore_barrier()
    pltpu.sync_copy(acc_sh.at[pl.ds(s * RPT, RPT)],
                    out_hbm.at[c, pl.ds(s * RPT, RPT)])


def _tc_scale_in(xp_ref, w1_ref, degp_ref, g1_ref, dis_ref):
    deg = 1.0 + degp_ref[0] + degp_ref[1]
    dis = lax.rsqrt(deg)
    h = jnp.dot(xp_ref[...], w1_ref[...], preferred_element_type=jnp.float32)
    dis_ref[...] = dis
    g1_ref[...] = h * dis[:, None]


def _tc_mid(s1_ref, g1_ref, dis_ref, b1_ref, g2_ref):
    dis = dis_ref[...]
    tot = s1_ref[0] + s1_ref[1] + g1_ref[...]
    r = jnp.maximum(tot * dis[:, None] + b1_ref[...][None, :], 0.0)
    g2_ref[...] = r * dis[:, None]


def _tc_out(s2_ref, g2_ref, dis_ref, w2_ref, b2_ref, o_ref):
    dis = dis_ref[...]
    agg = (s2_ref[0] + s2_ref[1] + g2_ref[...]) * dis[:, None]
    o = jnp.dot(agg, w2_ref[...], preferred_element_type=jnp.float32)
    o = o + b2_ref[...][None, :]
    m = jnp.max(o, axis=1, keepdims=True)
    lse = m + jnp.log(jnp.sum(jnp.exp(o - m), axis=1, keepdims=True))
    o_ref[...] = (o - lse)[:N]


def kernel(x, edge_index, W1, b1, W2, b2):
    xp = jnp.pad(x, ((0, NPAD - N), (0, 0)))

    degp = _sc_degree(edge_index)

    g1, dis = pl.pallas_call(
        _tc_scale_in,
        out_shape=(
            jax.ShapeDtypeStruct((NPAD, H), jnp.float32),
            jax.ShapeDtypeStruct((NPAD,), jnp.float32),
        ),
    )(xp, W1, degp)

    s1 = _sc_propagate(g1, edge_index)

    g2 = pl.pallas_call(
        _tc_mid,
        out_shape=jax.ShapeDtypeStruct((NPAD, H), jnp.float32),
    )(s1, g1, dis, b1)

    s2 = _sc_propagate(g2, edge_index)

    out = pl.pallas_call(
        _tc_out,
        out_shape=jax.ShapeDtypeStruct((N, C), jnp.float32),
    )(s2, g2, dis, W2, b2)

    return out


# speedup vs baseline: 97.8496x; 1.3642x over previous
import functools

import jax
import jax.numpy as jnp
from jax import lax
from jax.experimental import pallas as pl
from jax.experimental.pallas import tpu as pltpu
from jax.experimental.pallas import tpu_sc as plsc

N = 10000
E = 320000
D = 128
H = 16
C = 2

NC = 2
NS = 16
NW = NC * NS
EPT = E // NW
B = 128
NBLK = EPT // B
TAIL = EPT - NBLK * B
RING = 6
NPAD = 10240
RPT = NPAD // NS

_MESH = plsc.VectorSubcoreMesh(core_axis_name="c", subcore_axis_name="s")
_LANES = 16


@functools.partial(
    pl.kernel,
    out_type=jax.ShapeDtypeStruct((NC, NPAD), jnp.float32),
    mesh=_MESH,
    scratch_types=[
        pltpu.VMEM((EPT,), jnp.int32),
        pltpu.VMEM((B,), jnp.float32),
        pltpu.VMEM((RPT,), jnp.float32),
        pltpu.VMEM_SHARED((NPAD,), jnp.float32),
        pltpu.SemaphoreType.DMA,
    ],
    compiler_params=pltpu.CompilerParams(use_tc_tiling_on_sc=False),
)
def _sc_degree(edge_hbm, out_hbm, dst_v, ones_v, zv, acc_sh, sem):
    c = lax.axis_index("c")
    s = lax.axis_index("s")
    wid = s * NC + c

    for i in range(B // _LANES):
        ones_v[pl.ds(i * _LANES, _LANES)] = jnp.ones((_LANES,), jnp.float32)

    def zbody(i, _):
        zv[pl.ds(i * _LANES, _LANES)] = jnp.zeros((_LANES,), jnp.float32)
        return 0
    lax.fori_loop(0, RPT // _LANES, zbody, 0)

    pltpu.sync_copy(zv, acc_sh.at[pl.ds(s * RPT, RPT)])
    pltpu.sync_copy(edge_hbm.at[1, pl.ds(wid * EPT, EPT)], dst_v)
    plsc.subcore_barrier()

    def body(j, _):
        pltpu.async_copy(
            ones_v, acc_sh.at[dst_v.at[pl.ds(j * B, B)]], sem, add=True)

        @pl.when(j >= 2)
        def _wait():
            pltpu.make_async_copy(
                ones_v, acc_sh.at[dst_v.at[pl.ds(0, B)]], sem).wait()
        return 0
    lax.fori_loop(0, NBLK, body, 0)
    for _ in range(2):
        pltpu.make_async_copy(
            ones_v, acc_sh.at[dst_v.at[pl.ds(0, B)]], sem).wait()
    pltpu.sync_copy(ones_v.at[pl.ds(0, TAIL)],
                    acc_sh.at[dst_v.at[pl.ds(NBLK * B, TAIL)]], add=True)

    plsc.subcore_barrier()
    pltpu.sync_copy(acc_sh.at[pl.ds(s * RPT, RPT)],
                    out_hbm.at[c, pl.ds(s * RPT, RPT)])


@functools.partial(
    pl.kernel,
    out_type=jax.ShapeDtypeStruct((NC, NPAD, H), jnp.float32),
    mesh=_MESH,
    scratch_types=[
        pltpu.VMEM((EPT,), jnp.int32),
        pltpu.VMEM((EPT,), jnp.int32),
        pltpu.VMEM((RING, B, H), jnp.float32),
        pltpu.VMEM((TAIL, H), jnp.float32),
        pltpu.VMEM((RPT, H), jnp.float32),
        pltpu.VMEM_SHARED((NPAD, H), jnp.float32),
        [pltpu.SemaphoreType.DMA] * RING,
        [pltpu.SemaphoreType.DMA] * RING,
        pltpu.SemaphoreType.DMA,
    ],
    compiler_params=pltpu.CompilerParams(use_tc_tiling_on_sc=False),
)
def _sc_propagate(g_hbm, edge_hbm, out_hbm,
                  src_v, dst_v, rows_v, trow_v, zv, acc_sh, sg, ss, st):
    c = lax.axis_index("c")
    s = lax.axis_index("s")
    wid = s * NC + c

    def zbody(i, _):
        zv[i] = jnp.zeros((_LANES,), jnp.float32)
        return 0
    lax.fori_loop(0, RPT, zbody, 0)

    pltpu.sync_copy(zv, acc_sh.at[pl.ds(s * RPT, RPT)])
    pltpu.sync_copy(edge_hbm.at[0, pl.ds(wid * EPT, EPT)], src_v)
    pltpu.sync_copy(edge_hbm.at[1, pl.ds(wid * EPT, EPT)], dst_v)
    plsc.subcore_barrier()

    def gather_start(j, b):
        pltpu.async_copy(
            g_hbm.at[src_v.at[pl.ds(j * B, B)]], rows_v.at[b], sg[b])

    def gather_wait(b):
        pltpu.make_async_copy(
            g_hbm.at[src_v.at[pl.ds(0, B)]], rows_v.at[b], sg[b]).wait()

    def scatter_wait(b):
        pltpu.make_async_copy(
            rows_v.at[b], acc_sh.at[dst_v.at[pl.ds(0, B)]], ss[b]).wait()

    for b in range(RING - 1):
        gather_start(b, b)

    def body(i, _):
        for t in range(RING):
            j = i * RING + t
            nb = (t + RING - 1) % RING
            nj = j + RING - 1

            @pl.when(j > 0)
            def _ws():
                scatter_wait(nb)

            @pl.when(nj < NBLK)
            def _sg():
                gather_start(nj, nb)

            gather_wait(t)
            pltpu.async_copy(
                rows_v.at[t], acc_sh.at[dst_v.at[pl.ds(j * B, B)]],
                ss[t], add=True)
        return 0
    lax.fori_loop(0, NBLK // RING, body, 0)
    scatter_wait((NBLK - 1) % RING)

    pltpu.async_copy(
        g_hbm.at[src_v.at[pl.ds(NBLK * B, TAIL)]], trow_v, st).wait()
    pltpu.sync_copy(
        trow_v, acc_sh.at[dst_v.at[pl.ds(NBLK * B, TAIL)]], add=True)

    plsc.subcore_barrier()
    pltpu.sync_copy(acc_sh.at[pl.ds(s * RPT, RPT)],
                    out_hbm.at[c, pl.ds(s * RPT, RPT)])


PR = NPAD * H // 128


def _tc_scale_in(xp2_ref, w1_ref, degp_ref, g1_ref, disx_ref):
    deg8 = 1.0 + degp_ref[0] + degp_ref[1]
    dis8 = lax.rsqrt(deg8)
    a_i = lax.broadcasted_iota(jnp.int32, (8, 128), 0)
    a_j = lax.broadcasted_iota(jnp.int32, (8, 128), 1) // H
    b8 = jnp.where(a_i == a_j, 1.0, 0.0)
    disx = jnp.dot(dis8, b8, preferred_element_type=jnp.float32)
    w1t = jnp.tile(w1_ref[...], (8, 8))
    m_i = lax.broadcasted_iota(jnp.int32, (8 * D, 128), 0) // D
    m_j = lax.broadcasted_iota(jnp.int32, (8 * D, 128), 1) // H
    wbd = jnp.where(m_i == m_j, w1t, 0.0)
    hp = jnp.dot(xp2_ref[...], wbd, preferred_element_type=jnp.float32)
    disx_ref[...] = disx
    g1_ref[...] = hp * disx


def _tc_mid(s1_ref, g1_ref, disx_ref, b1_ref, g2_ref):
    disx = disx_ref[...]
    b1x = jnp.tile(b1_ref[...], 128 // H)
    tot = s1_ref[0] + s1_ref[1] + g1_ref[...]
    r = jnp.maximum(tot * disx + b1x[None, :], 0.0)
    g2_ref[...] = r * disx


def _tc_out(s2_ref, g2_ref, disx_ref, w2_ref, b2_ref, o_ref):
    aggp = (s2_ref[0] + s2_ref[1] + g2_ref[...]) * disx_ref[...]
    w2t = jnp.tile(w2_ref[...], (8, 8))
    m_i = lax.broadcasted_iota(jnp.int32, (128, 16), 0) // H
    m_j = lax.broadcasted_iota(jnp.int32, (128, 16), 1) // C
    wbd2 = jnp.where(m_i == m_j, w2t, 0.0)
    o = jnp.dot(aggp, wbd2, preferred_element_type=jnp.float32)
    b2x = jnp.tile(b2_ref[...], 16 // C)
    o = o + b2x[None, :]
    p_i = lax.broadcasted_iota(jnp.int32, (16, 16), 0) ^ 1
    p_j = lax.broadcasted_iota(jnp.int32, (16, 16), 1)
    perm = jnp.where(p_i == p_j, 1.0, 0.0)
    opart = jnp.dot(o, perm, preferred_element_type=jnp.float32)
    m = jnp.maximum(o, opart)
    lse = m + jnp.log(jnp.exp(o - m) + jnp.exp(opart - m))
    o_ref[...] = o - lse


def kernel(x, edge_index, W1, b1, W2, b2):
    xp2 = jnp.pad(x, ((0, NPAD - N), (0, 0))).reshape(PR, 8 * D)

    degp = _sc_degree(edge_index).reshape(NC, PR, 8)

    g1p, disx = pl.pallas_call(
        _tc_scale_in,
        out_shape=(
            jax.ShapeDtypeStruct((PR, 128), jnp.float32),
            jax.ShapeDtypeStruct((PR, 128), jnp.float32),
        ),
    )(xp2, W1, degp)

    s1 = _sc_propagate(g1p.reshape(NPAD, H), edge_index).reshape(NC, PR, 128)

    g2p = pl.pallas_call(
        _tc_mid,
        out_shape=jax.ShapeDtypeStruct((PR, 128), jnp.float32),
    )(s1, g1p, disx, b1)

    s2 = _sc_propagate(g2p.reshape(NPAD, H), edge_index).reshape(NC, PR, 128)

    op = pl.pallas_call(
        _tc_out,
        out_shape=jax.ShapeDtypeStruct((PR, 16), jnp.float32),
    )(s2, g2p, disx, W2, b2)

    return op.reshape(NPAD, C)[:N]
